# Initial kernel scaffold; baseline (speedup 1.0000x reference)
import jax
import jax.numpy as jnp
from jax.experimental import pallas as pl


def kernel(enc_emb, dec_emb, w_enc, b_enc, w_out, b_out, src, tgt):
    raise NotImplementedError("write your pallas kernel here")


# speedup vs baseline: 15.4733x; 15.4733x over previous
import jax
import jax.numpy as jnp
from jax import lax
from jax.experimental import pallas as pl
from jax.experimental.pallas import tpu as pltpu


def _attn_kernel(src_ref, tgt_ref, w_enc_ref, b_enc_ref, h_ref):
    ctx = jnp.tanh(
        jnp.dot(src_ref[...], w_enc_ref[...],
                preferred_element_type=jnp.float32)
        + b_enc_ref[...])

    e = tgt_ref[...]
    scores = lax.dot_general(
        e, ctx, (((1,), (1,)), ((), ())),
        preferred_element_type=jnp.float32)

    m = jnp.max(scores, axis=-1, keepdims=True)
    p = jnp.exp(scores - m)
    p = p / jnp.sum(p, axis=-1, keepdims=True)

    attn = jnp.dot(p, ctx, preferred_element_type=jnp.float32)
    h_ref[...] = (e + attn).astype(jnp.bfloat16)


def _proj_kernel(h_ref, w_ref, b_ref, o_ref):
    w = w_ref[...].astype(jnp.bfloat16)
    o_ref[...] = (
        jnp.dot(h_ref[...], w, preferred_element_type=jnp.float32)
        + b_ref[...])


def kernel(enc_emb, dec_emb, w_enc, b_enc, w_out, b_out, src, tgt):
    src_emb = enc_emb[src]
    tgt_emb = dec_emb[tgt]

    B, T_src, E = src_emb.shape
    _, T_tgt, H = tgt_emb.shape
    V = w_out.shape[1]

    h = pl.pallas_call(
        _attn_kernel,
        out_shape=jax.ShapeDtypeStruct((B * T_tgt, H), jnp.bfloat16),
        grid=(B,),
        in_specs=[
            pl.BlockSpec((pl.Squeezed(), T_src, E), lambda b: (b, 0, 0)),
            pl.BlockSpec((pl.Squeezed(), T_tgt, H), lambda b: (b, 0, 0)),
            pl.BlockSpec((E, H), lambda b: (0, 0)),
            pl.BlockSpec((1, H), lambda b: (0, 0)),
        ],
        out_specs=pl.BlockSpec((T_tgt, H), lambda b: (b, 0)),
        compiler_params=pltpu.CompilerParams(
            dimension_semantics=("parallel",)),
    )(src_emb, tgt_emb, w_enc, b_enc)

    tile_v = 512
    n_vt = V // tile_v
    logits = pl.pallas_call(
        _proj_kernel,
        out_shape=jax.ShapeDtypeStruct((B * T_tgt, V), jnp.float32),
        grid=(n_vt,),
        in_specs=[
            pl.BlockSpec((B * T_tgt, H), lambda j: (0, 0)),
            pl.BlockSpec((H, tile_v), lambda j: (0, j)),
            pl.BlockSpec((1, tile_v), lambda j: (0, j)),
        ],
        out_specs=pl.BlockSpec((B * T_tgt, tile_v), lambda j: (0, j)),
        compiler_params=pltpu.CompilerParams(
            dimension_semantics=("parallel",)),
    )(h, w_out, b_out)

    return logits.reshape(B, T_tgt, V)


# speedup vs baseline: 20.0942x; 1.2986x over previous
import jax
import jax.numpy as jnp
from jax import lax
from jax.experimental import pallas as pl
from jax.experimental.pallas import tpu as pltpu


def _make_kernel(n_rows, t_tgt):
    def _kernel(src_ref, tgt_ref, w_enc_ref, b_enc_ref, w_out_ref,
                b_out_ref, o_ref, ctx_ref, h_ref):
        j = pl.program_id(0)

        @pl.when(j == 0)
        def _():
            ctx_ref[...] = jnp.tanh(
                jnp.dot(src_ref[...], w_enc_ref[...],
                        preferred_element_type=jnp.float32)
                + b_enc_ref[...]).astype(jnp.bfloat16)

            for i in range(n_rows):
                sl = pl.ds(i * t_tgt, t_tgt)
                ctx = ctx_ref[sl, :]
                e = tgt_ref[sl, :]
                scores = lax.dot_general(
                    e.astype(jnp.bfloat16), ctx, (((1,), (1,)), ((), ())),
                    preferred_element_type=jnp.float32)
                m = jnp.max(scores, axis=-1, keepdims=True)
                p = jnp.exp(scores - m)
                p = p / jnp.sum(p, axis=-1, keepdims=True)
                attn = jnp.dot(p.astype(jnp.bfloat16), ctx,
                               preferred_element_type=jnp.float32)
                h_ref[sl, :] = (e + attn).astype(jnp.bfloat16)

        w = w_out_ref[...].astype(jnp.bfloat16)
        o_ref[...] = (
            jnp.dot(h_ref[...], w, preferred_element_type=jnp.float32)
            + b_out_ref[...])

    return _kernel


def kernel(enc_emb, dec_emb, w_enc, b_enc, w_out, b_out, src, tgt):
    src_emb = enc_emb[src]
    tgt_emb = dec_emb[tgt]

    B, T_src, E = src_emb.shape
    _, T_tgt, H = tgt_emb.shape
    V = w_out.shape[1]

    src_flat = src_emb.reshape(B * T_src, E)
    tgt_flat = tgt_emb.reshape(B * T_tgt, H)

    tile_v = min(512, V)
    n_vt = V // tile_v

    logits = pl.pallas_call(
        _make_kernel(B, T_tgt),
        out_shape=jax.ShapeDtypeStruct((B * T_tgt, V), jnp.float32),
        grid=(n_vt,),
        in_specs=[
            pl.BlockSpec((B * T_src, E), lambda j: (0, 0)),
            pl.BlockSpec((B * T_tgt, H), lambda j: (0, 0)),
            pl.BlockSpec((E, H), lambda j: (0, 0)),
            pl.BlockSpec((1, H), lambda j: (0, 0)),
            pl.BlockSpec((H, tile_v), lambda j: (0, j)),
            pl.BlockSpec((1, tile_v), lambda j: (0, j)),
        ],
        out_specs=pl.BlockSpec((B * T_tgt, tile_v), lambda j: (0, j)),
        scratch_shapes=[
            pltpu.VMEM((B * T_src, H), jnp.bfloat16),
            pltpu.VMEM((B * T_tgt, H), jnp.bfloat16),
        ],
        compiler_params=pltpu.CompilerParams(
            dimension_semantics=("arbitrary",)),
    )(src_flat, tgt_flat, w_enc, b_enc, w_out, b_out)

    return logits.reshape(B, T_tgt, V)


# speedup vs baseline: 20.5021x; 1.0203x over previous
import jax
import jax.numpy as jnp
from jax import lax
from jax.experimental import pallas as pl
from jax.experimental.pallas import tpu as pltpu


def _make_kernel(n_rows, t_tgt):
    def _kernel(src_ref, tgt_ref, w_enc_ref, b_enc_ref, w_out_ref,
                b_out_ref, o_ref, ctx_ref, h_ref):
        jv = pl.program_id(1)

        @pl.when(jv == 0)
        def _():
            ctx_ref[...] = jnp.tanh(
                jnp.dot(src_ref[...], w_enc_ref[...],
                        preferred_element_type=jnp.float32)
                + b_enc_ref[...]).astype(jnp.bfloat16)

            for i in range(n_rows):
                sl = pl.ds(i * t_tgt, t_tgt)
                ctx = ctx_ref[sl, :]
                e = tgt_ref[sl, :]
                scores = lax.dot_general(
                    e.astype(jnp.bfloat16), ctx, (((1,), (1,)), ((), ())),
                    preferred_element_type=jnp.float32)
                m = jnp.max(scores, axis=-1, keepdims=True)
                p = jnp.exp(scores - m)
                p = p / jnp.sum(p, axis=-1, keepdims=True)
                attn = jnp.dot(p.astype(jnp.bfloat16), ctx,
                               preferred_element_type=jnp.float32)
                h_ref[sl, :] = (e + attn).astype(jnp.bfloat16)

        w = w_out_ref[...].astype(jnp.bfloat16)
        o_ref[...] = (
            jnp.dot(h_ref[...], w, preferred_element_type=jnp.float32)
            + b_out_ref[...])

    return _kernel


def kernel(enc_emb, dec_emb, w_enc, b_enc, w_out, b_out, src, tgt):
    src_emb = enc_emb[src]
    tgt_emb = dec_emb[tgt]

    B, T_src, E = src_emb.shape
    _, T_tgt, H = tgt_emb.shape
    V = w_out.shape[1]

    src_flat = src_emb.reshape(B * T_src, E)
    tgt_flat = tgt_emb.reshape(B * T_tgt, H)

    tile_v = min(1024, V)
    n_vt = V // tile_v
    n_par = 2 if n_vt % 2 == 0 else 1
    n_seq = n_vt // n_par

    logits = pl.pallas_call(
        _make_kernel(B, T_tgt),
        out_shape=jax.ShapeDtypeStruct((B * T_tgt, V), jnp.float32),
        grid=(n_par, n_seq),
        in_specs=[
            pl.BlockSpec((B * T_src, E), lambda c, j: (0, 0)),
            pl.BlockSpec((B * T_tgt, H), lambda c, j: (0, 0)),
            pl.BlockSpec((E, H), lambda c, j: (0, 0)),
            pl.BlockSpec((1, H), lambda c, j: (0, 0)),
            pl.BlockSpec((H, tile_v), lambda c, j: (0, c * n_seq + j)),
            pl.BlockSpec((1, tile_v), lambda c, j: (0, c * n_seq + j)),
        ],
        out_specs=pl.BlockSpec(
            (B * T_tgt, tile_v), lambda c, j: (0, c * n_seq + j)),
        scratch_shapes=[
            pltpu.VMEM((B * T_src, H), jnp.bfloat16),
            pltpu.VMEM((B * T_tgt, H), jnp.bfloat16),
        ],
        compiler_params=pltpu.CompilerParams(
            dimension_semantics=("parallel", "arbitrary")),
    )(src_flat, tgt_flat, w_enc, b_enc, w_out, b_out)

    return logits.reshape(B, T_tgt, V)


# speedup vs baseline: 20.6907x; 1.0092x over previous
import jax
import jax.numpy as jnp
from jax import lax
from jax.experimental import pallas as pl
from jax.experimental.pallas import tpu as pltpu


def _make_kernel(n_rows, t_tgt):
    def _kernel(src_ref, tgt_ref, w_enc_ref, b_enc_ref, w_out_ref,
                b_out_ref, o_ref, ctx_ref, h_ref):
        j = pl.program_id(0)

        @pl.when(j == 0)
        def _():
            ctx_ref[...] = jnp.tanh(
                jnp.dot(src_ref[...], w_enc_ref[...],
                        preferred_element_type=jnp.float32)
                + b_enc_ref[...]).astype(jnp.bfloat16)

            for i in range(n_rows):
                sl = pl.ds(i * t_tgt, t_tgt)
                ctx = ctx_ref[sl, :]
                e = tgt_ref[sl, :]
                scores = lax.dot_general(
                    e.astype(jnp.bfloat16), ctx, (((1,), (1,)), ((), ())),
                    preferred_element_type=jnp.float32)
                m = jnp.max(scores, axis=-1, keepdims=True)
                p = jnp.exp(scores - m)
                p = p / jnp.sum(p, axis=-1, keepdims=True)
                attn = jnp.dot(p.astype(jnp.bfloat16), ctx,
                               preferred_element_type=jnp.float32)
                h_ref[sl, :] = (e + attn).astype(jnp.bfloat16)

        o_ref[...] = (
            jnp.dot(h_ref[...], w_out_ref[...],
                    preferred_element_type=jnp.float32)
            + b_out_ref[...])

    return _kernel


def kernel(enc_emb, dec_emb, w_enc, b_enc, w_out, b_out, src, tgt):
    src_emb = enc_emb[src]
    tgt_emb = dec_emb[tgt]
    w_out_bf = w_out.astype(jnp.bfloat16)

    B, T_src, E = src_emb.shape
    _, T_tgt, H = tgt_emb.shape
    V = w_out.shape[1]

    src_flat = src_emb.reshape(B * T_src, E)
    tgt_flat = tgt_emb.reshape(B * T_tgt, H)

    tile_v = min(1024, V)
    n_vt = V // tile_v

    logits = pl.pallas_call(
        _make_kernel(B, T_tgt),
        out_shape=jax.ShapeDtypeStruct((B * T_tgt, V), jnp.float32),
        grid=(n_vt,),
        in_specs=[
            pl.BlockSpec((B * T_src, E), lambda j: (0, 0)),
            pl.BlockSpec((B * T_tgt, H), lambda j: (0, 0)),
            pl.BlockSpec((E, H), lambda j: (0, 0)),
            pl.BlockSpec((1, H), lambda j: (0, 0)),
            pl.BlockSpec((H, tile_v), lambda j: (0, j)),
            pl.BlockSpec((1, tile_v), lambda j: (0, j)),
        ],
        out_specs=pl.BlockSpec((B * T_tgt, tile_v), lambda j: (0, j)),
        scratch_shapes=[
            pltpu.VMEM((B * T_src, H), jnp.bfloat16),
            pltpu.VMEM((B * T_tgt, H), jnp.bfloat16),
        ],
        compiler_params=pltpu.CompilerParams(
            dimension_semantics=("arbitrary",)),
    )(src_flat, tgt_flat, w_enc, b_enc, w_out_bf, b_out)

    return logits.reshape(B, T_tgt, V)


# speedup vs baseline: 21.4086x; 1.0347x over previous
import jax
import jax.numpy as jnp
from jax import lax
from jax.experimental import pallas as pl
from jax.experimental.pallas import tpu as pltpu


def _enc_kernel(src_ref, w_enc_ref, b_enc_ref, ctx_ref):
    ctx_ref[...] = jnp.tanh(
        jnp.dot(src_ref[...], w_enc_ref[...],
                preferred_element_type=jnp.float32)
        + b_enc_ref[...]).astype(jnp.bfloat16)


def _make_main_kernel(n_rows, t_tgt):
    def _kernel(tgt_ref, ctx_in_ref, w_out_ref, b_out_ref, o_ref, h_ref):
        j = pl.program_id(0)

        @pl.when(j == 0)
        def _():
            for i in range(n_rows):
                sl = pl.ds(i * t_tgt, t_tgt)
                ctx = ctx_in_ref[sl, :]
                e = tgt_ref[sl, :]
                scores = lax.dot_general(
                    e.astype(jnp.bfloat16), ctx, (((1,), (1,)), ((), ())),
                    preferred_element_type=jnp.float32)
                m = jnp.max(scores, axis=-1, keepdims=True)
                p = jnp.exp(scores - m)
                p = p / jnp.sum(p, axis=-1, keepdims=True)
                attn = jnp.dot(p.astype(jnp.bfloat16), ctx,
                               preferred_element_type=jnp.float32)
                h_ref[sl, :] = (e + attn).astype(jnp.bfloat16)

        w = w_out_ref[...].astype(jnp.bfloat16)
        o_ref[...] = (
            jnp.dot(h_ref[...], w, preferred_element_type=jnp.float32)
            + b_out_ref[...])

    return _kernel


def kernel(enc_emb, dec_emb, w_enc, b_enc, w_out, b_out, src, tgt):
    src_emb = enc_emb[src]
    tgt_emb = dec_emb[tgt]

    B, T_src, E = src_emb.shape
    _, T_tgt, H = tgt_emb.shape
    V = w_out.shape[1]

    src_flat = src_emb.reshape(B * T_src, E)
    tgt_flat = tgt_emb.reshape(B * T_tgt, H)

    ctx = pl.pallas_call(
        _enc_kernel,
        out_shape=jax.ShapeDtypeStruct((B * T_src, H), jnp.bfloat16),
        grid=(1,),
        in_specs=[
            pl.BlockSpec((B * T_src, E), lambda j: (0, 0)),
            pl.BlockSpec((E, H), lambda j: (0, 0)),
            pl.BlockSpec((1, H), lambda j: (0, 0)),
        ],
        out_specs=pl.BlockSpec((B * T_src, H), lambda j: (0, 0)),
        compiler_params=pltpu.CompilerParams(
            dimension_semantics=("arbitrary",)),
    )(src_flat, w_enc, b_enc)

    tile_v = min(2048, V)
    n_vt = V // tile_v

    logits = pl.pallas_call(
        _make_main_kernel(B, T_tgt),
        out_shape=jax.ShapeDtypeStruct((B * T_tgt, V), jnp.float32),
        grid=(n_vt,),
        in_specs=[
            pl.BlockSpec((B * T_tgt, H), lambda j: (0, 0)),
            pl.BlockSpec((B * T_src, H), lambda j: (0, 0)),
            pl.BlockSpec((H, tile_v), lambda j: (0, j)),
            pl.BlockSpec((1, tile_v), lambda j: (0, j)),
        ],
        out_specs=pl.BlockSpec((B * T_tgt, tile_v), lambda j: (0, j)),
        scratch_shapes=[
            pltpu.VMEM((B * T_tgt, H), jnp.bfloat16),
        ],
        compiler_params=pltpu.CompilerParams(
            dimension_semantics=("arbitrary",)),
    )(tgt_flat, ctx, w_out, b_out)

    return logits.reshape(B, T_tgt, V)
